# baseline (speedup 1.0000x reference)
import functools
import math

import jax
import jax.numpy as jnp
import numpy as np
from jax import lax
from jax.experimental import pallas as pl
from jax.experimental.pallas import tpu as pltpu
from jax.experimental.pallas import tpu_sc as plsc

N = 10000
E = 320000
IN_DIM = 128
OUT_DIM = 128
TIME_DIM = 32

NC = 2
NS = 16
NW = NC * NS
EPW = E // NW
C = 40
HS = (16, 24)
HOFF = (0, 16)
NCHUNK = EPW // C
PW = 144
BE = 2000
ROWS_PER_TILE = N // NS
ZROWS = 125

BN = 1000

_PERM128 = np.concatenate(
    [np.concatenate([np.arange(0, 32, 2), np.arange(1, 32, 2)]) + 32 * b
     for b in range(4)])
_PERM32 = np.concatenate([np.arange(0, 32, 2), np.arange(1, 32, 2)])


def _tables_body(h_ref, wvh_ref, wq_ref, wk_ref, ts_ref, td_ref):
    h = h_ref[...]
    hv = jnp.dot(h, wvh_ref[...], preferred_element_type=jnp.float32)
    ts_ref[...] = hv.astype(jnp.bfloat16)
    wqk = lax.dot_general(wq_ref[...], wk_ref[...], (((1,), (1,)), ((), ())),
                          preferred_element_type=jnp.float32)
    qk = jnp.dot(h, wqk, preferred_element_type=jnp.float32)
    td_ref[...] = qk.astype(jnp.bfloat16)


def _tev_body(te_ref, wvt_ref, tv_ref):
    tv_ref[...] = jnp.dot(te_ref[...], wvt_ref[...],
                          preferred_element_type=jnp.float32
                          ).astype(jnp.bfloat16)


def _lane_shuffle(v, idx):
    return lax.gather(
        v, idx[:, None],
        lax.GatherDimensionNumbers(offset_dims=(), collapsed_slice_dims=(0,),
                                   start_index_map=(0,)),
        (1,), mode=lax.GatherScatterMode.PROMISE_IN_BOUNDS)


def _lane_allsum(v, lane):
    for m in (8, 4, 2, 1):
        v = v + _lane_shuffle(v, lane ^ m)
    return v


def _unpack2(ref, j, off):
    return plsc.unpack(ref[j, pl.ds(off, 32)],
                       format=plsc.PackFormat.INTERLEAVED,
                       preferred_element_type=jnp.float32)


def _edge_body(ts, td, tv, ei, acc_out,
               idx0, idx1, sr0, sr1, dr0, dr1, tr0, tr1, pay0, pay1, accsp,
               ss0, sd0, st0, ss1, sd1, st1, sp0, sp1, si0, si1):
    cid = lax.axis_index("c")
    sid = lax.axis_index("s")
    wid = sid * NC + cid
    inv_scale = 1.0 / math.sqrt(float(OUT_DIM))
    lane = lax.broadcasted_iota(jnp.int32, (16,), 0)
    idxb = (idx0, idx1)
    srb = (sr0, sr1)
    drb = (dr0, dr1)
    trb = (tr0, tr1)
    payb = (pay0, pay1)
    spb = (sp0, sp1)
    sib = (si0, si1)
    sems = ((ss0, sd0, st0), (ss1, sd1, st1))

    zh = HS[0]
    def zrow(r, carry):
        for c in range(PW // 16):
            pay0[r, pl.ds(c * 16, 16)] = jnp.zeros((16,), jnp.float32)
        return carry
    lax.fori_loop(0, zh, zrow, 0)
    base_rows = sid * ROWS_PER_TILE
    for r in range(ROWS_PER_TILE // zh):
        pltpu.sync_copy(pay0, accsp.at[pl.ds(base_rows + r * zh, zh)])
    pltpu.sync_copy(pay0.at[pl.ds(0, ROWS_PER_TILE % zh)],
                    accsp.at[pl.ds(base_rows + zh * (ROWS_PER_TILE // zh),
                                   ROWS_PER_TILE % zh)])
    plsc.subcore_barrier()

    def idx_prefetch(b, chunk):
        base = wid * EPW + chunk * C
        pltpu.async_copy(ei.at[:, pl.ds(base, C)], idxb[b], sib[b])

    def idx_wait(b):
        pltpu.make_async_copy(ei.at[:, pl.ds(0, C)], idxb[b], sib[b]).wait()

    def fetch(b, chunk):
        base = wid * EPW + chunk * C
        idx_wait(b)
        pltpu.async_copy(ts.at[idxb[b].at[0]], srb[b], sems[b][0])
        pltpu.async_copy(td.at[idxb[b].at[1]], drb[b], sems[b][1])
        pltpu.async_copy(tv.at[pl.ds(base, C)], trb[b], sems[b][2])

    def compute_half(b, h):
        srows, drows, trows = srb[b], drb[b], trb[b]
        pay = payb[h]

        def edge(jo):
            j = HOFF[h] + jo
            me, mo = [], []
            for kk in range(4):
                a, bb = _unpack2(srows, j, kk * 32)
                t_a, t_b = _unpack2(trows, j, kk * 32)
                me.append(a + t_a)
                mo.append(bb + t_b)
            acc = None
            for kk in range(4):
                qe, qo = _unpack2(drows, j, kk * 32)
                p = me[kk] * qe + mo[kk] * qo
                acc = p if acc is None else acc + p
            exv = jnp.exp(_lane_allsum(acc, lane) * inv_scale)
            for kk in range(4):
                pay[jo, pl.ds(kk * 32, 16)] = exv * me[kk]
                pay[jo, pl.ds(kk * 32 + 16, 16)] = exv * mo[kk]
            pay[jo, pl.ds(128, 16)] = exv

        def edge4(j4, ecarry):
            for i in range(4):
                edge(j4 * 4 + i)
            return ecarry
        lax.fori_loop(0, HS[h] // 4, edge4, 0)

    def scatter_wait(b, h):
        pltpu.make_async_copy(payb[h],
                              accsp.at[idxb[b].at[1, pl.ds(HOFF[h], HS[h])]],
                              spb[h]).wait()

    idx_prefetch(0, 0)
    fetch(0, 0)
    idx_prefetch(1, 1)

    def outer(k2, carry):
        for b in range(2):
            k = k2 * 2 + b
            pltpu.make_async_copy(ts.at[idxb[b].at[0]], srb[b],
                                  sems[b][0]).wait()
            pltpu.make_async_copy(td.at[idxb[b].at[1]], drb[b],
                                  sems[b][1]).wait()
            pltpu.make_async_copy(tv.at[pl.ds(0, C)], trb[b],
                                  sems[b][2]).wait()
            if b == 0:
                fetch(1, k + 1)
            else:
                @pl.when(k2 < NCHUNK // 2 - 1)
                def _():
                    fetch(0, k + 1)
            for h in range(2):
                if b == 1:
                    scatter_wait(b ^ 1, h)
                else:
                    @pl.when(k2 > 0)
                    def _():
                        scatter_wait(b ^ 1, h)
                compute_half(b, h)
                pltpu.async_copy(payb[h],
                                 accsp.at[idxb[b].at[1,
                                                     pl.ds(HOFF[h], HS[h])]],
                                 spb[h], add=True)
            @pl.when(k2 < NCHUNK // 2 - 1)
            def _():
                idx_prefetch(b, k + 2)
        return carry
    lax.fori_loop(0, NCHUNK // 2, outer, 0)
    for h in range(2):
        scatter_wait(1, h)

    plsc.subcore_barrier()
    for r in range(ROWS_PER_TILE // ZROWS):
        off = base_rows + r * ZROWS
        pltpu.sync_copy(accsp.at[pl.ds(off, ZROWS)],
                        acc_out.at[cid, pl.ds(off, ZROWS)])


def _final_body(acc_ref, h_ref, wop_ref, wself_ref, bo_ref, bself_ref,
                o_ref):
    acc = acc_ref[0] + acc_ref[1]
    sh = acc[:, :128]
    d = acc[:, 128:129]
    inv = jnp.where(d > 0.0, 1.0 / d, 0.0)
    pre = jnp.dot(sh, wop_ref[...], preferred_element_type=jnp.float32)
    o = (pre * inv
         + jnp.dot(h_ref[...], wself_ref[...],
                   preferred_element_type=jnp.float32)
         + bo_ref[...] + bself_ref[...])
    o_ref[...] = jnp.maximum(o, 0.0)


def kernel(h, edge_index, time_enc, W_v, W_k, W_q, W_o, b_o, W_self, b_self):
    wvh = W_v[:IN_DIM]
    wvt = W_v[IN_DIM:]
    wo_perm = W_o[_PERM128]

    tab_s, tab_d = pl.pallas_call(
        _tables_body,
        grid=(N // BN,),
        in_specs=[
            pl.BlockSpec((BN, IN_DIM), lambda i: (i, 0)),
            pl.BlockSpec((IN_DIM, OUT_DIM), lambda i: (0, 0)),
            pl.BlockSpec((IN_DIM, OUT_DIM), lambda i: (0, 0)),
            pl.BlockSpec((OUT_DIM, OUT_DIM), lambda i: (0, 0)),
        ],
        out_specs=[
            pl.BlockSpec((BN, 128), lambda i: (i, 0)),
            pl.BlockSpec((BN, 128), lambda i: (i, 0)),
        ],
        out_shape=[
            jax.ShapeDtypeStruct((N, 128), jnp.bfloat16),
            jax.ShapeDtypeStruct((N, 128), jnp.bfloat16),
        ],
    )(h, wvh, W_q, W_k)

    tev = pl.pallas_call(
        _tev_body,
        grid=(E // BE,),
        in_specs=[
            pl.BlockSpec((BE, TIME_DIM), lambda i: (i, 0)),
            pl.BlockSpec((TIME_DIM, OUT_DIM), lambda i: (0, 0)),
        ],
        out_specs=pl.BlockSpec((BE, OUT_DIM), lambda i: (i, 0)),
        out_shape=jax.ShapeDtypeStruct((E, OUT_DIM), jnp.bfloat16),
    )(time_enc, wvt)

    edge_kernel = functools.partial(
        pl.kernel,
        mesh=plsc.VectorSubcoreMesh(core_axis_name="c", subcore_axis_name="s"),
        out_type=jax.ShapeDtypeStruct((NC, N, PW), jnp.float32),
        scratch_types=[
            pltpu.VMEM((2, C), jnp.int32),
            pltpu.VMEM((2, C), jnp.int32),
            pltpu.VMEM((C, 128), jnp.bfloat16),
            pltpu.VMEM((C, 128), jnp.bfloat16),
            pltpu.VMEM((C, 128), jnp.bfloat16),
            pltpu.VMEM((C, 128), jnp.bfloat16),
            pltpu.VMEM((C, 128), jnp.bfloat16),
            pltpu.VMEM((C, 128), jnp.bfloat16),
            pltpu.VMEM((HS[0], PW), jnp.float32),
            pltpu.VMEM((HS[1], PW), jnp.float32),
            pltpu.VMEM_SHARED((N, PW), jnp.float32),
            pltpu.SemaphoreType.DMA,
            pltpu.SemaphoreType.DMA,
            pltpu.SemaphoreType.DMA,
            pltpu.SemaphoreType.DMA,
            pltpu.SemaphoreType.DMA,
            pltpu.SemaphoreType.DMA,
            pltpu.SemaphoreType.DMA,
            pltpu.SemaphoreType.DMA,
            pltpu.SemaphoreType.DMA,
            pltpu.SemaphoreType.DMA,
        ],
        compiler_params=pltpu.CompilerParams(use_tc_tiling_on_sc=False,
                                             needs_layout_passes=False),
    )(_edge_body)
    acc = edge_kernel(tab_s, tab_d, tev, edge_index)

    return pl.pallas_call(
        _final_body,
        grid=(N // BN,),
        in_specs=[
            pl.BlockSpec((NC, BN, PW), lambda i: (0, i, 0)),
            pl.BlockSpec((BN, IN_DIM), lambda i: (i, 0)),
            pl.BlockSpec((OUT_DIM, OUT_DIM), lambda i: (0, 0)),
            pl.BlockSpec((IN_DIM, OUT_DIM), lambda i: (0, 0)),
            pl.BlockSpec((1, OUT_DIM), lambda i: (0, 0)),
            pl.BlockSpec((1, OUT_DIM), lambda i: (0, 0)),
        ],
        out_specs=pl.BlockSpec((BN, OUT_DIM), lambda i: (i, 0)),
        out_shape=jax.ShapeDtypeStruct((N, OUT_DIM), jnp.float32),
    )(acc, h, wo_perm, W_self, b_o.reshape(1, -1), b_self.reshape(1, -1))


# speedup vs baseline: 1.4011x; 1.4011x over previous
import functools
import math

import jax
import jax.numpy as jnp
import numpy as np
from jax import lax
from jax.experimental import pallas as pl
from jax.experimental.pallas import tpu as pltpu
from jax.experimental.pallas import tpu_sc as plsc

N = 10000
E = 320000
IN_DIM = 128
OUT_DIM = 128
TIME_DIM = 32

NC = 2
NS = 16
NW = NC * NS
EPW = E // NW
C = 40
HS = (16, 24)
HOFF = (0, 16)
NCHUNK = EPW // C
PW = 176
ROWS_PER_TILE = N // NS
ZROWS = 125

BN = 1000

_PERM128 = np.concatenate(
    [np.concatenate([np.arange(0, 32, 2), np.arange(1, 32, 2)]) + 32 * b
     for b in range(4)])
_PERM32 = np.concatenate([np.arange(0, 32, 2), np.arange(1, 32, 2)])


def _tables_body(h_ref, wvh_ref, wvt_ref, wq_ref, wk_ref, ts_ref, td_ref):
    h = h_ref[...]
    hv = jnp.dot(h, wvh_ref[...], preferred_element_type=jnp.float32)
    ts_ref[...] = hv.astype(jnp.bfloat16)
    wqk = lax.dot_general(wq_ref[...], wk_ref[...], (((1,), (1,)), ((), ())),
                          preferred_element_type=jnp.float32)
    qk = jnp.dot(h, wqk, preferred_element_type=jnp.float32)
    qt = lax.dot_general(qk, wvt_ref[...], (((1,), (1,)), ((), ())),
                         preferred_element_type=jnp.float32)
    td_ref[...] = jnp.concatenate([qk, qt], axis=1).astype(jnp.bfloat16)


def _lane_shuffle(v, idx):
    return lax.gather(
        v, idx[:, None],
        lax.GatherDimensionNumbers(offset_dims=(), collapsed_slice_dims=(0,),
                                   start_index_map=(0,)),
        (1,), mode=lax.GatherScatterMode.PROMISE_IN_BOUNDS)


def _lane_allsum(v, lane):
    for m in (8, 4, 2, 1):
        v = v + _lane_shuffle(v, lane ^ m)
    return v


def _unpack2(ref, j, off):
    return plsc.unpack(ref[j, pl.ds(off, 32)],
                       format=plsc.PackFormat.INTERLEAVED,
                       preferred_element_type=jnp.float32)


def _edge_body(ts, td, te, ei, acc_out,
               idx0, idx1, sr0, sr1, dr0, dr1, tr0, tr1, pay0, pay1, accsp,
               ss0, sd0, st0, ss1, sd1, st1, sp0, sp1, si0, si1):
    cid = lax.axis_index("c")
    sid = lax.axis_index("s")
    wid = sid * NC + cid
    inv_scale = 1.0 / math.sqrt(float(OUT_DIM))
    lane = lax.broadcasted_iota(jnp.int32, (16,), 0)
    idxb = (idx0, idx1)
    srb = (sr0, sr1)
    drb = (dr0, dr1)
    trb = (tr0, tr1)
    payb = (pay0, pay1)
    spb = (sp0, sp1)
    sib = (si0, si1)
    sems = ((ss0, sd0, st0), (ss1, sd1, st1))

    zh = HS[0]
    def zrow(r, carry):
        for c in range(PW // 16):
            pay0[r, pl.ds(c * 16, 16)] = jnp.zeros((16,), jnp.float32)
        return carry
    lax.fori_loop(0, zh, zrow, 0)
    base_rows = sid * ROWS_PER_TILE
    for r in range(ROWS_PER_TILE // zh):
        pltpu.sync_copy(pay0, accsp.at[pl.ds(base_rows + r * zh, zh)])
    pltpu.sync_copy(pay0.at[pl.ds(0, ROWS_PER_TILE % zh)],
                    accsp.at[pl.ds(base_rows + zh * (ROWS_PER_TILE // zh),
                                   ROWS_PER_TILE % zh)])
    plsc.subcore_barrier()

    def idx_prefetch(b, chunk):
        base = wid * EPW + chunk * C
        pltpu.async_copy(ei.at[:, pl.ds(base, C)], idxb[b], sib[b])

    def idx_wait(b):
        pltpu.make_async_copy(ei.at[:, pl.ds(0, C)], idxb[b], sib[b]).wait()

    def fetch(b, chunk):
        base = wid * EPW + chunk * C
        idx_wait(b)
        pltpu.async_copy(ts.at[idxb[b].at[0]], srb[b], sems[b][0])
        pltpu.async_copy(td.at[idxb[b].at[1]], drb[b], sems[b][1])
        pltpu.async_copy(te.at[pl.ds(base, C)], trb[b], sems[b][2])

    def compute_half(b, h):
        srows, drows, trows = srb[b], drb[b], trb[b]
        pay = payb[h]

        def edge(jo):
            j = HOFF[h] + jo
            se, so = [], []
            for kk in range(4):
                a, bb = _unpack2(srows, j, kk * 32)
                se.append(a)
                so.append(bb)
            acc = None
            for kk in range(4):
                qe, qo = _unpack2(drows, j, kk * 32)
                p = se[kk] * qe + so[kk] * qo
                acc = p if acc is None else acc + p
            t_e, t_o = _unpack2(trows, j, 0)
            qte, qto = _unpack2(drows, j, 128)
            acc = acc + t_e * qte + t_o * qto
            exv = jnp.exp(_lane_allsum(acc, lane) * inv_scale)
            for kk in range(4):
                pay[jo, pl.ds(kk * 32, 16)] = exv * se[kk]
                pay[jo, pl.ds(kk * 32 + 16, 16)] = exv * so[kk]
            pay[jo, pl.ds(128, 16)] = exv * t_e
            pay[jo, pl.ds(144, 16)] = exv * t_o
            pay[jo, pl.ds(160, 16)] = exv

        def edge4(j4, ecarry):
            for i in range(4):
                edge(j4 * 4 + i)
            return ecarry
        lax.fori_loop(0, HS[h] // 4, edge4, 0)

    def scatter_wait(b, h):
        pltpu.make_async_copy(payb[h],
                              accsp.at[idxb[b].at[1, pl.ds(HOFF[h], HS[h])]],
                              spb[h]).wait()

    idx_prefetch(0, 0)
    fetch(0, 0)
    idx_prefetch(1, 1)

    def outer(k2, carry):
        for b in range(2):
            k = k2 * 2 + b
            pltpu.make_async_copy(ts.at[idxb[b].at[0]], srb[b],
                                  sems[b][0]).wait()
            pltpu.make_async_copy(td.at[idxb[b].at[1]], drb[b],
                                  sems[b][1]).wait()
            pltpu.make_async_copy(te.at[pl.ds(0, C)], trb[b],
                                  sems[b][2]).wait()
            if b == 0:
                fetch(1, k + 1)
            else:
                @pl.when(k2 < NCHUNK // 2 - 1)
                def _():
                    fetch(0, k + 1)
            for h in range(2):
                if b == 1:
                    scatter_wait(b ^ 1, h)
                else:
                    @pl.when(k2 > 0)
                    def _():
                        scatter_wait(b ^ 1, h)
                compute_half(b, h)
                pltpu.async_copy(payb[h],
                                 accsp.at[idxb[b].at[1,
                                                     pl.ds(HOFF[h], HS[h])]],
                                 spb[h], add=True)
            @pl.when(k2 < NCHUNK // 2 - 1)
            def _():
                idx_prefetch(b, k + 2)
        return carry
    lax.fori_loop(0, NCHUNK // 2, outer, 0)
    for h in range(2):
        scatter_wait(1, h)

    plsc.subcore_barrier()
    for r in range(ROWS_PER_TILE // ZROWS):
        off = base_rows + r * ZROWS
        pltpu.sync_copy(accsp.at[pl.ds(off, ZROWS)],
                        acc_out.at[cid, pl.ds(off, ZROWS)])


def _final_body(acc_ref, h_ref, wvtp_ref, wop_ref, wo_ref, wself_ref,
                bo_ref, bself_ref, o_ref):
    acc = acc_ref[0] + acc_ref[1]
    sh = acc[:, :128]
    st = acc[:, 128:160]
    d = acc[:, 160:161]
    inv = jnp.where(d > 0.0, 1.0 / d, 0.0)
    wvtwo = jnp.dot(wvtp_ref[...], wo_ref[...],
                    preferred_element_type=jnp.float32)
    pre = (jnp.dot(sh, wop_ref[...], preferred_element_type=jnp.float32)
           + jnp.dot(st, wvtwo, preferred_element_type=jnp.float32))
    o = (pre * inv
         + jnp.dot(h_ref[...], wself_ref[...],
                   preferred_element_type=jnp.float32)
         + bo_ref[...] + bself_ref[...])
    o_ref[...] = jnp.maximum(o, 0.0)


def kernel(h, edge_index, time_enc, W_v, W_k, W_q, W_o, b_o, W_self, b_self):
    wvh = W_v[:IN_DIM]
    wvt = W_v[IN_DIM:]
    te_bf = time_enc.astype(jnp.bfloat16)
    wvt_perm = wvt[_PERM32]
    wo_perm = W_o[_PERM128]

    tab_s, tab_d = pl.pallas_call(
        _tables_body,
        grid=(N // BN,),
        in_specs=[
            pl.BlockSpec((BN, IN_DIM), lambda i: (i, 0)),
            pl.BlockSpec((IN_DIM, OUT_DIM), lambda i: (0, 0)),
            pl.BlockSpec((TIME_DIM, OUT_DIM), lambda i: (0, 0)),
            pl.BlockSpec((IN_DIM, OUT_DIM), lambda i: (0, 0)),
            pl.BlockSpec((OUT_DIM, OUT_DIM), lambda i: (0, 0)),
        ],
        out_specs=[
            pl.BlockSpec((BN, 128), lambda i: (i, 0)),
            pl.BlockSpec((BN, 160), lambda i: (i, 0)),
        ],
        out_shape=[
            jax.ShapeDtypeStruct((N, 128), jnp.bfloat16),
            jax.ShapeDtypeStruct((N, 160), jnp.bfloat16),
        ],
    )(h, wvh, wvt, W_q, W_k)

    edge_kernel = functools.partial(
        pl.kernel,
        mesh=plsc.VectorSubcoreMesh(core_axis_name="c", subcore_axis_name="s"),
        out_type=jax.ShapeDtypeStruct((NC, N, PW), jnp.float32),
        scratch_types=[
            pltpu.VMEM((2, C), jnp.int32),
            pltpu.VMEM((2, C), jnp.int32),
            pltpu.VMEM((C, 128), jnp.bfloat16),
            pltpu.VMEM((C, 128), jnp.bfloat16),
            pltpu.VMEM((C, 160), jnp.bfloat16),
            pltpu.VMEM((C, 160), jnp.bfloat16),
            pltpu.VMEM((C, TIME_DIM), jnp.bfloat16),
            pltpu.VMEM((C, TIME_DIM), jnp.bfloat16),
            pltpu.VMEM((HS[0], PW), jnp.float32),
            pltpu.VMEM((HS[1], PW), jnp.float32),
            pltpu.VMEM_SHARED((N, PW), jnp.float32),
            pltpu.SemaphoreType.DMA,
            pltpu.SemaphoreType.DMA,
            pltpu.SemaphoreType.DMA,
            pltpu.SemaphoreType.DMA,
            pltpu.SemaphoreType.DMA,
            pltpu.SemaphoreType.DMA,
            pltpu.SemaphoreType.DMA,
            pltpu.SemaphoreType.DMA,
            pltpu.SemaphoreType.DMA,
            pltpu.SemaphoreType.DMA,
        ],
        compiler_params=pltpu.CompilerParams(use_tc_tiling_on_sc=False,
                                             needs_layout_passes=False),
    )(_edge_body)
    acc = edge_kernel(tab_s, tab_d, te_bf, edge_index)

    return pl.pallas_call(
        _final_body,
        grid=(N // BN,),
        in_specs=[
            pl.BlockSpec((NC, BN, PW), lambda i: (0, i, 0)),
            pl.BlockSpec((BN, IN_DIM), lambda i: (i, 0)),
            pl.BlockSpec((TIME_DIM, OUT_DIM), lambda i: (0, 0)),
            pl.BlockSpec((OUT_DIM, OUT_DIM), lambda i: (0, 0)),
            pl.BlockSpec((OUT_DIM, OUT_DIM), lambda i: (0, 0)),
            pl.BlockSpec((IN_DIM, OUT_DIM), lambda i: (0, 0)),
            pl.BlockSpec((1, OUT_DIM), lambda i: (0, 0)),
            pl.BlockSpec((1, OUT_DIM), lambda i: (0, 0)),
        ],
        out_specs=pl.BlockSpec((BN, OUT_DIM), lambda i: (i, 0)),
        out_shape=jax.ShapeDtypeStruct((N, OUT_DIM), jnp.float32),
    )(acc, h, wvt_perm, wo_perm, W_o, W_self,
      b_o.reshape(1, -1), b_self.reshape(1, -1))
